# baseline (speedup 1.0000x reference)
import functools
import jax
import jax.numpy as jnp
from jax import lax
from jax.experimental import pallas as pl
from jax.experimental.pallas import tpu as pltpu
from jax.experimental.pallas import tpu_sc as plsc

N_USERS = 25000
N_ITEMS = 25000
N = N_USERS + N_ITEMS
D = 64
DH = 32
N_LAYERS = 3
E = 800000

NC = 2
NS = 16
L = 16

C = 128
G = 8
BLOCKS = 50
CHUNKS = G * BLOCKS
E_TILE = C * CHUNKS
E_PAD = E_TILE * NS
EROWS = E_PAD // C

N_PAD = 50048
STRIPE = N_PAD // NS
MROWS = 136
NSEG = N_LAYERS + 1

_mesh = plsc.VectorSubcoreMesh(core_axis_name="c", subcore_axis_name="s")


@functools.partial(
    pl.kernel,
    out_type=(
        jax.ShapeDtypeStruct((NC * N_PAD, DH), jnp.float32),
        jax.ShapeDtypeStruct((NSEG * NC * N_PAD, DH), jnp.float32),
    ),
    mesh=_mesh,
    scratch_types=[
        pltpu.VMEM_SHARED((N_PAD, DH), jnp.float32),
        pltpu.VMEM((G, C), jnp.int32),
        pltpu.VMEM((G, C), jnp.int32),
        pltpu.VMEM((G, C), jnp.float32),
        pltpu.VMEM((G, C), jnp.int32),
        pltpu.VMEM((G, C), jnp.int32),
        pltpu.VMEM((G, C), jnp.float32),
        pltpu.VMEM((C * L,), jnp.int32),
        pltpu.VMEM((C, DH), jnp.float32),
        pltpu.VMEM((C, DH), jnp.float32),
        pltpu.VMEM((MROWS, DH), jnp.float32),
        pltpu.VMEM((MROWS, DH), jnp.float32),
        pltpu.SemaphoreType.DMA,
        pltpu.SemaphoreType.DMA,
        pltpu.SemaphoreType.DMA,
        pltpu.SemaphoreType.DMA,
        pltpu.SemaphoreType.DMA,
        pltpu.SemaphoreType.DMA,
    ],
    compiler_params=pltpu.CompilerParams(use_tc_tiling_on_sc=False,
                                         needs_layout_passes=False),
)
def _lightgcn_kernel(tab, dst_h, src_h, w_h, bidx_h, final, states,
                     acc, dstbA, srcbA, wbA, dstbB, srcbB, wbB, bidx_v,
                     rows0, rows1, macc, mld,
                     esemA, esemB, gsem0, gsem1, ssem0, ssem1):
    c = lax.axis_index("c")
    s = lax.axis_index("s")
    row0 = s * STRIPE
    rows_ = (rows0, rows1)
    gsems = (gsem0, gsem1)
    ssems = (ssem0, ssem1)
    bufA = (dstbA, srcbA, wbA)
    bufB = (dstbB, srcbB, wbB)
    esems = {id(bufA): esemA, id(bufB): esemB}

    def _drain_scatter(i):
        pltpu.make_async_copy(tab.at[pl.ds(0, C), :], rows_[i], ssems[i]).wait()

    def _wait_gather(i):
        pltpu.make_async_copy(tab.at[pl.ds(0, C), :], rows_[i], gsems[i]).wait()

    def _issue_edges(b, buf):
        rb = s * CHUNKS + b * G
        sem = esems[id(buf)]
        pltpu.async_copy(dst_h.at[pl.ds(rb, G), :], buf[0], sem)
        pltpu.async_copy(src_h.at[pl.ds(rb, G), :], buf[1], sem)
        pltpu.async_copy(w_h.at[pl.ds(rb, G), :], buf[2], sem)

    def _wait_edges(buf):
        sem = esems[id(buf)]
        for r in range(3):
            pltpu.make_async_copy(dst_h.at[pl.ds(0, G), :], buf[0], sem).wait()

    def _rebase(buf, goff):
        srcn = buf[1]

        @plsc.parallel_loop(0, C // L, step=1, unroll=2)
        def _rb(k):
            for q in range(G):
                sl = pl.ds(k * L, L)
                srcn[q, sl] = srcn[q, sl] + goff

    pltpu.sync_copy(bidx_h, bidx_v)

    def _fill(i, _):
        macc[i, 0:L] = jnp.zeros((L,), jnp.float32)
        macc[i, L:DH] = jnp.zeros((L,), jnp.float32)
        return 0
    lax.fori_loop(0, MROWS, _fill, 0)

    def _seed(j, _):
        r0 = row0 + j * MROWS
        pltpu.sync_copy(tab.at[pl.ds(c * N_PAD + r0, MROWS), :], mld)
        pltpu.sync_copy(mld, states.at[pl.ds(c * N_PAD + r0, MROWS), :])
        return 0
    lax.fori_loop(0, STRIPE // MROWS, _seed, 0)

    def _layer(l, _):
        def _zero(z, _):
            pltpu.sync_copy(macc, acc.at[pl.ds(row0 + z * MROWS, MROWS), :])
            return 0
        lax.fori_loop(0, STRIPE // MROWS, _zero, 0)
        plsc.subcore_barrier()

        goff = (l * NC + c) * N_PAD

        def _process(b, cur, nxt, first):
            dstc, srcc, wc = cur
            if first:
                @pl.when(b > 0)
                def _():
                    _drain_scatter(1)
            else:
                _drain_scatter(1)

            @pl.when(b + 1 < BLOCKS)
            def _():
                _issue_edges(b + 1, nxt)

            for q in range(G):
                bq = q % 2
                if q + 1 < G:
                    if q > 0:
                        _drain_scatter(1 - bq)
                    pltpu.async_copy(
                        states.at[srcc.at[q + 1]], rows_[1 - bq], gsems[1 - bq])
                else:
                    @pl.when(b + 1 < BLOCKS)
                    def _():
                        _wait_edges(nxt)
                        _rebase(nxt, goff)
                        _drain_scatter(0)
                        pltpu.async_copy(states.at[nxt[1].at[0]], rows0, gsem0)
                _wait_gather(bq)
                rv = rows_[bq]
                wq = wc.at[q]

                @plsc.parallel_loop(0, C, step=8, unroll=4)
                def _mul(e0):
                    for e8 in range(8):
                        e = e0 + e8
                        we = plsc.load_gather(wq, [bidx_v[pl.ds(e * L, L)]])
                        rv[e, 0:L] = rv[e, 0:L] * we
                        rv[e, L:DH] = rv[e, L:DH] * we
                pltpu.async_copy(rv, acc.at[dstc.at[q]], ssems[bq], add=True)

        _issue_edges(0, bufA)
        _wait_edges(bufA)
        _rebase(bufA, goff)
        pltpu.async_copy(states.at[srcbA.at[0]], rows0, gsem0)

        def _pair(p, _):
            _process(2 * p, bufA, bufB, first=True)
            _process(2 * p + 1, bufB, bufA, first=False)
            return 0

        lax.fori_loop(0, BLOCKS // 2, _pair, 0)
        _drain_scatter(0)
        _drain_scatter(1)
        plsc.subcore_barrier()

        woff = ((l + 1) * NC + c) * N_PAD
        pltpu.sync_copy(acc.at[pl.ds(row0, STRIPE), :],
                        states.at[pl.ds(woff + row0, STRIPE), :])
        return 0

    lax.fori_loop(0, N_LAYERS, _layer, 0)

    def _meanchunk(j, _):
        r0 = row0 + j * MROWS
        pltpu.sync_copy(tab.at[pl.ds(c * N_PAD + r0, MROWS), :], macc)
        for l in range(N_LAYERS):
            pltpu.sync_copy(
                states.at[pl.ds(((l + 1) * NC + c) * N_PAD + r0, MROWS), :], mld)

            def _acc(r, _):
                macc[r, 0:L] = macc[r, 0:L] + mld[r, 0:L]
                macc[r, L:DH] = macc[r, L:DH] + mld[r, L:DH]
                return 0
            lax.fori_loop(0, MROWS, _acc, 0)

        def _scale(r, _):
            macc[r, 0:L] = macc[r, 0:L] * 0.25
            macc[r, L:DH] = macc[r, L:DH] * 0.25
            return 0
        lax.fori_loop(0, MROWS, _scale, 0)
        pltpu.sync_copy(macc, final.at[pl.ds(c * N_PAD + r0, MROWS), :])
        return 0

    lax.fori_loop(0, STRIPE // MROWS, _meanchunk, 0)


def kernel(edge_index, edge_weight, user_emb, item_emb):
    all_emb = jnp.concatenate([user_emb, item_emb], axis=0)
    pad_rows = jnp.zeros((N_PAD - N, DH), jnp.float32)
    tab = jnp.concatenate([all_emb[:, :DH], pad_rows,
                           all_emb[:, DH:], pad_rows], axis=0)

    npad = E_PAD - E
    dst = jnp.concatenate([edge_index[0], jnp.full((npad,), N, jnp.int32)])
    src = jnp.concatenate([edge_index[1], jnp.zeros((npad,), jnp.int32)])
    w = jnp.concatenate([edge_weight, jnp.zeros((npad,), jnp.float32)])
    dst2 = dst.reshape(EROWS, C)
    src2 = src.reshape(EROWS, C)
    w2 = w.reshape(EROWS, C)

    bidx = jnp.repeat(jnp.arange(C, dtype=jnp.int32), L)
    final, _ = _lightgcn_kernel(tab, dst2, src2, w2, bidx)
    user = jnp.concatenate([final[:N_USERS], final[N_PAD:N_PAD + N_USERS]], axis=1)
    item = jnp.concatenate([final[N_USERS:N], final[N_PAD + N_USERS:N_PAD + N]], axis=1)
    return (user, item)


# speedup vs baseline: 1.2283x; 1.2283x over previous
import functools
import jax
import jax.numpy as jnp
from jax import lax
from jax.experimental import pallas as pl
from jax.experimental.pallas import tpu as pltpu
from jax.experimental.pallas import tpu_sc as plsc

N_USERS = 25000
N_ITEMS = 25000
N = N_USERS + N_ITEMS
D = 64
DH = 32
N_LAYERS = 3
E = 800000

NC = 2
NS = 16
L = 16

C = 128
G = 8
BLOCKS = 49
CHUNKS = G * BLOCKS
E_TILE = C * CHUNKS
E_PAD = E_TILE * NS
EROWS = E_PAD // C

N_PAD = 50048
STRIPE = N_PAD // NS
MROWS = 136

_mesh = plsc.VectorSubcoreMesh(core_axis_name="c", subcore_axis_name="s")


@functools.partial(
    pl.kernel,
    out_type=(
        jax.ShapeDtypeStruct((NC * N_PAD, DH), jnp.float32),
        jax.ShapeDtypeStruct((N_LAYERS * NC * N_PAD, DH), jnp.float32),
    ),
    mesh=_mesh,
    scratch_types=[
        pltpu.VMEM_SHARED((N_PAD, DH), jnp.float32),
        pltpu.VMEM((G, C), jnp.int32),
        pltpu.VMEM((G, C), jnp.int32),
        pltpu.VMEM((G, C), jnp.float32),
        pltpu.VMEM((C * L,), jnp.int32),
        pltpu.VMEM((C, DH), jnp.float32),
        pltpu.VMEM((C, DH), jnp.float32),
        pltpu.VMEM((MROWS, DH), jnp.float32),
        pltpu.VMEM((MROWS, DH), jnp.float32),
        pltpu.SemaphoreType.DMA,
        pltpu.SemaphoreType.DMA,
        pltpu.SemaphoreType.DMA,
        pltpu.SemaphoreType.DMA,
        pltpu.SemaphoreType.DMA,
    ],
    compiler_params=pltpu.CompilerParams(use_tc_tiling_on_sc=False,
                                         needs_layout_passes=False),
)
def _lightgcn_kernel(tab, dst_h, src_h, w_h, bidx_h, final, layers,
                     acc, dstb, srcb, wb, bidx_v, rows0, rows1,
                     macc, mld, esem, gsem0, gsem1, ssem0, ssem1):
    c = lax.axis_index("c")
    s = lax.axis_index("s")
    row0 = s * STRIPE
    rows_ = (rows0, rows1)
    gsems = (gsem0, gsem1)
    ssems = (ssem0, ssem1)

    def _drain_scatter(i):
        pltpu.make_async_copy(tab.at[pl.ds(0, C), :], rows_[i], ssems[i]).wait()

    pltpu.sync_copy(bidx_h, bidx_v)

    def _fill(i, _):
        macc[i, 0:L] = jnp.zeros((L,), jnp.float32)
        macc[i, L:DH] = jnp.zeros((L,), jnp.float32)
        return 0
    lax.fori_loop(0, MROWS, _fill, 0)

    for l in range(N_LAYERS):
        def _zero(z, _):
            pltpu.sync_copy(macc, acc.at[pl.ds(row0 + z * MROWS, MROWS), :])
            return 0
        lax.fori_loop(0, STRIPE // MROWS, _zero, 0)
        plsc.subcore_barrier()

        if l == 0:
            src_tab = tab
            goff = c * N_PAD
        else:
            src_tab = layers
            goff = ((l - 1) * NC + c) * N_PAD

        def _block(b, _):
            rbase = s * CHUNKS + b * G
            d1 = pltpu.async_copy(dst_h.at[pl.ds(rbase, G), :], dstb, esem)
            d2 = pltpu.async_copy(src_h.at[pl.ds(rbase, G), :], srcb, esem)
            d3 = pltpu.async_copy(w_h.at[pl.ds(rbase, G), :], wb, esem)
            d1.wait(); d2.wait(); d3.wait()
            @plsc.parallel_loop(0, C // L, step=1, unroll=2)
            def _rebase(k):
                for q in range(G):
                    sl = pl.ds(k * L, L)
                    srcb[q, sl] = srcb[q, sl] + goff

            @pl.when(b > 0)
            def _():
                _drain_scatter(0)
            gd = [None, None]
            gd[0] = pltpu.async_copy(src_tab.at[srcb.at[0]], rows0, gsem0)
            for q in range(G):
                bq = q % 2
                if q + 1 < G:
                    if q == 0:
                        @pl.when(b > 0)
                        def _():
                            _drain_scatter(1)
                    else:
                        _drain_scatter(1 - bq)
                    gd[1 - bq] = pltpu.async_copy(
                        src_tab.at[srcb.at[q + 1]], rows_[1 - bq], gsems[1 - bq])
                gd[bq].wait()
                rv = rows_[bq]
                wq = wb.at[q]

                @plsc.parallel_loop(0, C, step=8, unroll=4)
                def _mul(e0):
                    for e8 in range(8):
                        e = e0 + e8
                        we = plsc.load_gather(wq, [bidx_v[pl.ds(e * L, L)]])
                        rv[e, 0:L] = rv[e, 0:L] * we
                        rv[e, L:DH] = rv[e, L:DH] * we
                pltpu.async_copy(rv, acc.at[dstb.at[q]], ssems[bq], add=True)
            return 0

        lax.fori_loop(0, BLOCKS, _block, 0)
        _drain_scatter(0)
        _drain_scatter(1)
        plsc.subcore_barrier()

        woff = (l * NC + c) * N_PAD
        pltpu.sync_copy(acc.at[pl.ds(row0, STRIPE), :],
                        layers.at[pl.ds(woff + row0, STRIPE), :])

    def _meanchunk(j, _):
        r0 = row0 + j * MROWS
        pltpu.sync_copy(tab.at[pl.ds(c * N_PAD + r0, MROWS), :], macc)
        for l in range(N_LAYERS):
            pltpu.sync_copy(
                layers.at[pl.ds((l * NC + c) * N_PAD + r0, MROWS), :], mld)

            def _acc(r, _):
                macc[r, 0:L] = macc[r, 0:L] + mld[r, 0:L]
                macc[r, L:DH] = macc[r, L:DH] + mld[r, L:DH]
                return 0
            lax.fori_loop(0, MROWS, _acc, 0)

        def _scale(r, _):
            macc[r, 0:L] = macc[r, 0:L] * 0.25
            macc[r, L:DH] = macc[r, L:DH] * 0.25
            return 0
        lax.fori_loop(0, MROWS, _scale, 0)
        pltpu.sync_copy(macc, final.at[pl.ds(c * N_PAD + r0, MROWS), :])
        return 0

    lax.fori_loop(0, STRIPE // MROWS, _meanchunk, 0)


def kernel(edge_index, edge_weight, user_emb, item_emb):
    all_emb = jnp.concatenate([user_emb, item_emb], axis=0)
    pad_rows = jnp.zeros((N_PAD - N, DH), jnp.float32)
    tab = jnp.concatenate([all_emb[:, :DH], pad_rows,
                           all_emb[:, DH:], pad_rows], axis=0)

    npad = E_PAD - E
    dst = jnp.concatenate([edge_index[0], jnp.full((npad,), N, jnp.int32)])
    src = jnp.concatenate([edge_index[1], jnp.zeros((npad,), jnp.int32)])
    w = jnp.concatenate([edge_weight, jnp.zeros((npad,), jnp.float32)])
    dst2 = dst.reshape(EROWS, C)
    src2 = src.reshape(EROWS, C)
    w2 = w.reshape(EROWS, C)

    bidx = jnp.repeat(jnp.arange(C, dtype=jnp.int32), L)
    final, _ = _lightgcn_kernel(tab, dst2, src2, w2, bidx)
    user = jnp.concatenate([final[:N_USERS], final[N_PAD:N_PAD + N_USERS]], axis=1)
    item = jnp.concatenate([final[N_USERS:N], final[N_PAD + N_USERS:N_PAD + N]], axis=1)
    return (user, item)
